# baseline (speedup 1.0000x reference)
import functools

import jax
import jax.numpy as jnp
from jax import lax
from jax.experimental import pallas as pl
from jax.experimental.pallas import tpu as pltpu
from jax.experimental.pallas import tpu_sc as plsc

N, K, F, R = 10000, 32, 128, 128
E = N * K

NS = 16
NW = NS

NSPLIT = 10
NH = N // NSPLIT
EH = NH * K

CHUNK = 80
CPW = EH // (CHUNK * NW)
assert CPW * CHUNK * NW == EH and (CPW * CHUNK) % 8 == 0 and CHUNK % 8 == 0


def _softplus(z):
    return jnp.maximum(z, 0.0) + jnp.log1p(jnp.exp(-jnp.abs(z)))


def _v_body(x_ref, w_ref, b_ref, o_ref):
    z = jnp.dot(x_ref[...], w_ref[...], preferred_element_type=jnp.float32)
    o_ref[...] = _softplus(z + b_ref[...])


def _atom_layer1(x, W1, b1):
    blk = 2000
    return pl.pallas_call(
        _v_body,
        grid=(N // blk,),
        in_specs=[
            pl.BlockSpec((blk, F), lambda i: (i, 0)),
            pl.BlockSpec((F, F), lambda i: (0, 0)),
            pl.BlockSpec((1, F), lambda i: (0, 0)),
        ],
        out_specs=pl.BlockSpec((blk, F), lambda i: (i, 0)),
        out_shape=jax.ShapeDtypeStruct((N, F), jnp.float32),
    )(x, W1, b1.reshape(1, F))


SETS = 2
SETSZ = 5
NBUF = SETS * SETSZ
NG = CPW // SETSZ
FINAL_SEL = ((NG - 1) % 2) * SETSZ


def _sc_gather_body(table_hbm, idx_hbm, out_hbm, idx_v, rows_v, gsem, ssem):
    base = lax.axis_index("s") * CPW
    pltpu.sync_copy(idx_hbm.at[pl.ds(base * CHUNK, CPW * CHUNK)], idx_v)

    def group(g, carry):
        sel = (g % 2) * SETSZ
        for b in range(SETSZ):
            pltpu.async_copy(
                table_hbm.at[idx_v.at[pl.ds((g * SETSZ + b) * CHUNK, CHUNK)]],
                rows_v.at[pl.ds((sel + b) * CHUNK, CHUNK)], gsem)
        @pl.when(g > 0)
        def _():
            osel = SETSZ - sel
            pltpu.make_async_copy(
                rows_v.at[pl.ds(osel * CHUNK, SETSZ * CHUNK)],
                out_hbm.at[pl.ds((base + (g - 1) * SETSZ) * CHUNK,
                                 SETSZ * CHUNK)],
                ssem).wait()

        for b in range(SETSZ):
            pltpu.make_async_copy(
                table_hbm.at[idx_v.at[pl.ds((g * SETSZ + b) * CHUNK, CHUNK)]],
                rows_v.at[pl.ds((sel + b) * CHUNK, CHUNK)], gsem).wait()
        pltpu.async_copy(
            rows_v.at[pl.ds(sel * CHUNK, SETSZ * CHUNK)],
            out_hbm.at[pl.ds((base + g * SETSZ) * CHUNK, SETSZ * CHUNK)],
            ssem)
        return carry

    lax.fori_loop(0, NG, group, 0)
    pltpu.make_async_copy(
        rows_v.at[pl.ds(FINAL_SEL * CHUNK, SETSZ * CHUNK)],
        out_hbm.at[pl.ds((base + (NG - 1) * SETSZ) * CHUNK, SETSZ * CHUNK)],
        ssem).wait()


@functools.cache
def _make_sc_gather():
    return pl.kernel(
        _sc_gather_body,
        out_type=jax.ShapeDtypeStruct((EH, F), jnp.float32),
        mesh=plsc.VectorSubcoreMesh(core_axis_name="c", subcore_axis_name="s",
                                    num_cores=1),
        scratch_types=[
            pltpu.VMEM((CPW * CHUNK,), jnp.int32),
            pltpu.VMEM((NBUF * CHUNK, F), jnp.float32),
            pltpu.SemaphoreType.DMA,
            pltpu.SemaphoreType.DMA,
        ],
    )


BN = 200
BE = BN * K


def _main_body(rbf_ref, nbr_ref, x_ref, wf1_ref, bf1_ref, wf2_ref, bf2_ref,
               w2_ref, b2_ref, o_ref):
    h = _softplus(
        jnp.dot(rbf_ref[...].astype(jnp.bfloat16), wf1_ref[...],
                preferred_element_type=jnp.float32)
        + bf1_ref[...])
    filt = (jnp.dot(h.astype(jnp.bfloat16), wf2_ref[...],
                    preferred_element_type=jnp.float32)
            + bf2_ref[...])
    prod = filt * nbr_ref[...]
    agg = prod.reshape(BN, K, F).sum(axis=1)
    v2 = _softplus(
        jnp.dot(agg, w2_ref[...], preferred_element_type=jnp.float32)
        + b2_ref[...])
    o_ref[...] = x_ref[...] + v2


@functools.cache
def _make_cfconv_out(piece):
    off = piece * (NH // BN)

    return pl.pallas_call(
        _main_body,
        grid=(NH // BN,),
        in_specs=[
            pl.BlockSpec((BE, R), lambda i: (i + off, 0)),
            pl.BlockSpec((BE, F), lambda i: (i, 0)),
            pl.BlockSpec((BN, F), lambda i: (i + off, 0)),
            pl.BlockSpec((R, F), lambda i: (0, 0)),
            pl.BlockSpec((1, F), lambda i: (0, 0)),
            pl.BlockSpec((F, F), lambda i: (0, 0)),
            pl.BlockSpec((1, F), lambda i: (0, 0)),
            pl.BlockSpec((F, F), lambda i: (0, 0)),
            pl.BlockSpec((1, F), lambda i: (0, 0)),
        ],
        out_specs=pl.BlockSpec((BN, F), lambda i: (i, 0)),
        out_shape=jax.ShapeDtypeStruct((NH, F), jnp.float32),
    )


def _cfconv_out(piece, rbf_flat, nbr, x, Wf1, bf1, Wf2, bf2, W2, b2):
    return _make_cfconv_out(piece)(
        rbf_flat, nbr, x,
        Wf1.astype(jnp.bfloat16), bf1.reshape(1, F),
        Wf2.astype(jnp.bfloat16), bf2.reshape(1, F),
        W2, b2.reshape(1, F))


@jax.jit
def kernel(x, rbf, neighbors, W1, b1, Wf1, bf1, Wf2, bf2, W2, b2):
    v = _atom_layer1(x, W1, b1)
    rbf_flat = rbf.reshape(E, R)
    flat_nb = neighbors.reshape(-1)
    gather = _make_sc_gather()
    nbrs = [
        gather(v, lax.dynamic_slice_in_dim(flat_nb, p * EH, EH))
        for p in range(NSPLIT)
    ]
    outs = [
        _cfconv_out(p, rbf_flat, nbrs[p], x, Wf1, bf1, Wf2, bf2, W2, b2)
        for p in range(NSPLIT)
    ]
    return jnp.concatenate(outs, axis=0)


# speedup vs baseline: 1.0724x; 1.0724x over previous
import functools

import jax
import jax.numpy as jnp
from jax import lax
from jax.experimental import pallas as pl
from jax.experimental.pallas import tpu as pltpu
from jax.experimental.pallas import tpu_sc as plsc

N, K, F, R = 10000, 32, 128, 128
E = N * K

NS = 16
NW = NS

NSPLIT = 5
NH = N // NSPLIT
EH = NH * K

CHUNK = 80
CPW = EH // (CHUNK * NW)
assert CPW * CHUNK * NW == EH and (CPW * CHUNK) % 8 == 0 and CHUNK % 8 == 0


def _softplus(z):
    return jnp.maximum(z, 0.0) + jnp.log1p(jnp.exp(-jnp.abs(z)))


def _v_body(x_ref, w_ref, b_ref, o_ref):
    z = jnp.dot(x_ref[...], w_ref[...], preferred_element_type=jnp.float32)
    o_ref[...] = _softplus(z + b_ref[...])


def _atom_layer1(x, W1, b1):
    blk = 2000
    return pl.pallas_call(
        _v_body,
        grid=(N // blk,),
        in_specs=[
            pl.BlockSpec((blk, F), lambda i: (i, 0)),
            pl.BlockSpec((F, F), lambda i: (0, 0)),
            pl.BlockSpec((1, F), lambda i: (0, 0)),
        ],
        out_specs=pl.BlockSpec((blk, F), lambda i: (i, 0)),
        out_shape=jax.ShapeDtypeStruct((N, F), jnp.float32),
    )(x, W1, b1.reshape(1, F))


SETS = 2
SETSZ = 5
NBUF = SETS * SETSZ
NG = CPW // SETSZ
FINAL_SEL = ((NG - 1) % 2) * SETSZ


def _sc_gather_body(table_hbm, idx_hbm, out_hbm, idx_v, rows_v, gsem, ssem):
    base = lax.axis_index("s") * CPW
    pltpu.sync_copy(idx_hbm.at[pl.ds(base * CHUNK, CPW * CHUNK)], idx_v)

    def group(g, carry):
        sel = (g % 2) * SETSZ
        for b in range(SETSZ):
            pltpu.async_copy(
                table_hbm.at[idx_v.at[pl.ds((g * SETSZ + b) * CHUNK, CHUNK)]],
                rows_v.at[pl.ds((sel + b) * CHUNK, CHUNK)], gsem)
        @pl.when(g > 0)
        def _():
            osel = SETSZ - sel
            pltpu.make_async_copy(
                rows_v.at[pl.ds(osel * CHUNK, SETSZ * CHUNK)],
                out_hbm.at[pl.ds((base + (g - 1) * SETSZ) * CHUNK,
                                 SETSZ * CHUNK)],
                ssem).wait()

        for b in range(SETSZ):
            pltpu.make_async_copy(
                table_hbm.at[idx_v.at[pl.ds((g * SETSZ + b) * CHUNK, CHUNK)]],
                rows_v.at[pl.ds((sel + b) * CHUNK, CHUNK)], gsem).wait()
        pltpu.async_copy(
            rows_v.at[pl.ds(sel * CHUNK, SETSZ * CHUNK)],
            out_hbm.at[pl.ds((base + g * SETSZ) * CHUNK, SETSZ * CHUNK)],
            ssem)
        return carry

    lax.fori_loop(0, NG, group, 0)
    pltpu.make_async_copy(
        rows_v.at[pl.ds(FINAL_SEL * CHUNK, SETSZ * CHUNK)],
        out_hbm.at[pl.ds((base + (NG - 1) * SETSZ) * CHUNK, SETSZ * CHUNK)],
        ssem).wait()


@functools.cache
def _make_sc_gather():
    return pl.kernel(
        _sc_gather_body,
        out_type=jax.ShapeDtypeStruct((EH, F), jnp.float32),
        mesh=plsc.VectorSubcoreMesh(core_axis_name="c", subcore_axis_name="s",
                                    num_cores=1),
        scratch_types=[
            pltpu.VMEM((CPW * CHUNK,), jnp.int32),
            pltpu.VMEM((NBUF * CHUNK, F), jnp.float32),
            pltpu.SemaphoreType.DMA,
            pltpu.SemaphoreType.DMA,
        ],
    )


BN = 400
BE = BN * K


def _main_body(rbf_ref, nbr_ref, x_ref, wf1_ref, bf1_ref, wf2_ref, bf2_ref,
               w2_ref, b2_ref, o_ref):
    h = _softplus(
        jnp.dot(rbf_ref[...].astype(jnp.bfloat16), wf1_ref[...],
                preferred_element_type=jnp.float32)
        + bf1_ref[...])
    filt = (jnp.dot(h.astype(jnp.bfloat16), wf2_ref[...],
                    preferred_element_type=jnp.float32)
            + bf2_ref[...])
    prod = filt * nbr_ref[...]
    agg = prod.reshape(BN, K, F).sum(axis=1)
    v2 = _softplus(
        jnp.dot(agg, w2_ref[...], preferred_element_type=jnp.float32)
        + b2_ref[...])
    o_ref[...] = x_ref[...] + v2


@functools.cache
def _make_cfconv_out(piece):
    off = piece * (NH // BN)

    return pl.pallas_call(
        _main_body,
        grid=(NH // BN,),
        in_specs=[
            pl.BlockSpec((BE, R), lambda i: (i + off, 0)),
            pl.BlockSpec((BE, F), lambda i: (i, 0)),
            pl.BlockSpec((BN, F), lambda i: (i + off, 0)),
            pl.BlockSpec((R, F), lambda i: (0, 0)),
            pl.BlockSpec((1, F), lambda i: (0, 0)),
            pl.BlockSpec((F, F), lambda i: (0, 0)),
            pl.BlockSpec((1, F), lambda i: (0, 0)),
            pl.BlockSpec((F, F), lambda i: (0, 0)),
            pl.BlockSpec((1, F), lambda i: (0, 0)),
        ],
        out_specs=pl.BlockSpec((BN, F), lambda i: (i, 0)),
        out_shape=jax.ShapeDtypeStruct((NH, F), jnp.float32),
    )


def _cfconv_out(piece, rbf_flat, nbr, x, Wf1, bf1, Wf2, bf2, W2, b2):
    return _make_cfconv_out(piece)(
        rbf_flat, nbr, x,
        Wf1.astype(jnp.bfloat16), bf1.reshape(1, F),
        Wf2.astype(jnp.bfloat16), bf2.reshape(1, F),
        W2, b2.reshape(1, F))


@jax.jit
def kernel(x, rbf, neighbors, W1, b1, Wf1, bf1, Wf2, bf2, W2, b2):
    v = _atom_layer1(x, W1, b1)
    rbf_flat = rbf.reshape(E, R)
    flat_nb = neighbors.reshape(-1)
    gather = _make_sc_gather()
    nbrs = [
        gather(v, lax.dynamic_slice_in_dim(flat_nb, p * EH, EH))
        for p in range(NSPLIT)
    ]
    outs = [
        _cfconv_out(p, rbf_flat, nbrs[p], x, Wf1, bf1, Wf2, bf2, W2, b2)
        for p in range(NSPLIT)
    ]
    return jnp.concatenate(outs, axis=0)
